# initial kernel scaffold (unmeasured)
import jax
import jax.numpy as jnp
from jax import lax
from jax.experimental import pallas as pl
from jax.experimental.pallas import tpu as pltpu

N_Y = 4
M = 8192
D = 2048
CHUNK = M // N_Y
W = 1024
N_T = D // W
EPS = 1e-6


def kernel(partial, gamma):
    x = partial.reshape(M, D)
    g = gamma.reshape(1, D)

    def body(x_ref, g_ref, out_ref,
             send0, load, comm, send_sems, recv_sems, load_sem, credit_sem):
        my_x = lax.axis_index("x")
        my_y = lax.axis_index("y")
        my_z = lax.axis_index("z")
        right = lax.rem(my_y + 1, N_Y)
        left = lax.rem(my_y + N_Y - 1, N_Y)

        barrier = pltpu.get_barrier_semaphore()
        for nbr in (left, right):
            pl.semaphore_signal(
                barrier, inc=1,
                device_id=(my_x, nbr, my_z),
                device_id_type=pl.DeviceIdType.MESH,
            )
        pl.semaphore_wait(barrier, 2)

        def load_chunk(c, t, dst):
            cp = pltpu.make_async_copy(
                x_ref.at[pl.ds(c * CHUNK, CHUNK), pl.ds(t * W, W)],
                dst, load_sem)
            cp.start()
            cp.wait()

        for t in range(N_T):
            if t > 0:
                pl.semaphore_wait(credit_sem, 1)

            load_chunk(lax.rem(my_y + N_Y - 1, N_Y), t, send0)

            for h in range(N_Y - 1):
                src = send0 if h == 0 else comm.at[h - 1]
                rdma = pltpu.make_async_remote_copy(
                    src_ref=src,
                    dst_ref=comm.at[h],
                    send_sem=send_sems.at[h],
                    recv_sem=recv_sems.at[h],
                    device_id=(my_x, right, my_z),
                    device_id_type=pl.DeviceIdType.MESH,
                )
                rdma.start()
                rdma.wait()
                if h < N_Y - 2:
                    load_chunk(lax.rem(my_y + 2 * N_Y - 2 - h, N_Y), t, load)
                    comm[h, :, :] = comm[h, :, :] + load[:, :]

            load_chunk(my_y, t, load)
            out_ref[:, pl.ds(t * W, W)] = comm[N_Y - 2, :, :] + load[:, :]

            pl.semaphore_signal(
                credit_sem, inc=1,
                device_id=(my_x, left, my_z),
                device_id_type=pl.DeviceIdType.MESH,
            )

        pl.semaphore_wait(credit_sem, 1)

        y = out_ref[:, :]
        rms = jnp.sqrt(jnp.mean(y * y, axis=1, keepdims=True) + EPS)
        out_ref[:, :] = y / rms * g_ref[:, :]

    return pl.pallas_call(
        body,
        out_shape=jax.ShapeDtypeStruct((CHUNK, D), jnp.float32),
        in_specs=[
            pl.BlockSpec(memory_space=pltpu.ANY),
            pl.BlockSpec(memory_space=pltpu.VMEM),
        ],
        out_specs=pl.BlockSpec(memory_space=pltpu.VMEM),
        scratch_shapes=[
            pltpu.VMEM((CHUNK, W), jnp.float32),
            pltpu.VMEM((CHUNK, W), jnp.float32),
            pltpu.VMEM((N_Y - 1, CHUNK, W), jnp.float32),
            pltpu.SemaphoreType.DMA((N_Y - 1,)),
            pltpu.SemaphoreType.DMA((N_Y - 1,)),
            pltpu.SemaphoreType.DMA,
            pltpu.SemaphoreType.REGULAR,
        ],
        compiler_params=pltpu.CompilerParams(collective_id=0),
    )(x, g)


# baseline (device time: 609562 ns/iter reference)
import jax
import jax.numpy as jnp
from jax import lax
from jax.experimental import pallas as pl
from jax.experimental.pallas import tpu as pltpu

N_Y = 4
M = 8192
D = 2048
CHUNK = M // N_Y
W = 1024
N_T = D // W
EPS = 1e-6


def kernel(partial, gamma):
    x = partial.reshape(M, D)
    g = gamma.reshape(1, D)

    def body(x_ref, g_ref, out_ref,
             send0, load, comm, send_sems, recv_sems, load_sem, credit_sem):
        my_x = lax.axis_index("x")
        my_y = lax.axis_index("y")
        my_z = lax.axis_index("z")
        right = lax.rem(my_y + 1, N_Y)
        left = lax.rem(my_y + N_Y - 1, N_Y)

        barrier = pltpu.get_barrier_semaphore()
        for nbr in (left, right):
            pl.semaphore_signal(
                barrier, inc=1,
                device_id=(my_x, nbr, my_z),
                device_id_type=pl.DeviceIdType.MESH,
            )
        pl.semaphore_wait(barrier, 2)

        def load_chunk(c, t, dst):
            cp = pltpu.make_async_copy(
                x_ref.at[pl.ds(c * CHUNK, CHUNK), pl.ds(t * W, W)],
                dst, load_sem)
            cp.start()
            cp.wait()

        for t in range(N_T):
            if t > 0:
                pl.semaphore_wait(credit_sem, 1)

            load_chunk(lax.rem(my_y + N_Y - 1, N_Y), t, send0)

            for h in range(N_Y - 1):
                src = send0 if h == 0 else comm.at[h - 1]
                rdma = pltpu.make_async_remote_copy(
                    src_ref=src,
                    dst_ref=comm.at[h],
                    send_sem=send_sems.at[h],
                    recv_sem=recv_sems.at[h],
                    device_id=(my_x, right, my_z),
                    device_id_type=pl.DeviceIdType.MESH,
                )
                rdma.start()
                rdma.wait()
                if h < N_Y - 2:
                    load_chunk(lax.rem(my_y + 2 * N_Y - 2 - h, N_Y), t, load)
                    comm[h, :, :] = comm[h, :, :] + load[:, :]

            load_chunk(my_y, t, load)
            out_ref[:, pl.ds(t * W, W)] = comm[N_Y - 2, :, :] + load[:, :]

            pl.semaphore_signal(
                credit_sem, inc=1,
                device_id=(my_x, left, my_z),
                device_id_type=pl.DeviceIdType.MESH,
            )

        pl.semaphore_wait(credit_sem, 1)

        y = out_ref[:, :]
        rms = jnp.sqrt(jnp.mean(y * y, axis=1, keepdims=True) + EPS)
        out_ref[:, :] = y / rms * g_ref[:, :]

    return pl.pallas_call(
        body,
        out_shape=jax.ShapeDtypeStruct((CHUNK, D), jnp.float32),
        in_specs=[
            pl.BlockSpec(memory_space=pl.ANY),
            pl.BlockSpec(memory_space=pltpu.VMEM),
        ],
        out_specs=pl.BlockSpec(memory_space=pltpu.VMEM),
        scratch_shapes=[
            pltpu.VMEM((CHUNK, W), jnp.float32),
            pltpu.VMEM((CHUNK, W), jnp.float32),
            pltpu.VMEM((N_Y - 1, CHUNK, W), jnp.float32),
            pltpu.SemaphoreType.DMA((N_Y - 1,)),
            pltpu.SemaphoreType.DMA((N_Y - 1,)),
            pltpu.SemaphoreType.DMA,
            pltpu.SemaphoreType.REGULAR,
        ],
        compiler_params=pltpu.CompilerParams(
            collective_id=0,
            vmem_limit_bytes=100 * 1024 * 1024,
        ),
    )(x, g)
